# Initial kernel scaffold; baseline (speedup 1.0000x reference)
import jax
import jax.numpy as jnp
from jax.experimental import pallas as pl


def kernel(input_ids, emb_table, W1, b1, W2, b2):
    raise NotImplementedError("write your pallas kernel here")


# speedup vs baseline: 2.1959x; 2.1959x over previous
import functools

import jax
import jax.numpy as jnp
from jax import lax
from jax.experimental import pallas as pl
from jax.experimental.pallas import tpu as pltpu
from jax.experimental.pallas import tpu_sc as plsc


def _sc_gather(ids_flat, emb_table):
    n = ids_flat.shape[0]
    d = emb_table.shape[1]
    info = plsc.get_sparse_core_info()
    nw = info.num_cores * info.num_subcores
    rows_per_w = n // nw
    ch = 16
    n_ch = rows_per_w // ch
    mesh = plsc.VectorSubcoreMesh(core_axis_name="c", subcore_axis_name="s")

    @functools.partial(
        pl.kernel,
        mesh=mesh,
        out_type=jax.ShapeDtypeStruct((n, d), jnp.float32),
        scratch_types=[
            pltpu.VMEM((rows_per_w,), jnp.int32),
            pltpu.VMEM((ch, d), jnp.float32),
            pltpu.SemaphoreType.DMA,
        ],
    )
    def gather_k(table_hbm, idx_hbm, out_hbm, idx_v, buf, sem):
        wid = lax.axis_index("s") * info.num_cores + lax.axis_index("c")
        base = wid * rows_per_w
        pltpu.sync_copy(idx_hbm.at[pl.ds(base, rows_per_w)], idx_v)

        def body(c, carry):
            pltpu.async_copy(
                table_hbm.at[idx_v.at[pl.ds(c * ch, ch)]], buf, sem
            ).wait()
            pltpu.sync_copy(buf, out_hbm.at[pl.ds(base + c * ch, ch)])
            return carry

        lax.fori_loop(0, n_ch, body, 0)

    return gather_k(emb_table, ids_flat)


def _tc_mlp(x, W1, b1, W2, b2):
    n, d = x.shape
    bm = 512

    def mlp_body(x_ref, w1_ref, b1_ref, w2_ref, b2_ref, o_ref):
        xv = x_ref[...]
        h = jnp.dot(xv, w1_ref[...], preferred_element_type=jnp.float32)
        h = h + b1_ref[...]
        g = 0.5 * h * (1.0 + lax.erf(h * 0.7071067811865476))
        o = jnp.dot(g, w2_ref[...], preferred_element_type=jnp.float32)
        o_ref[...] = xv + o + b2_ref[...]

    return pl.pallas_call(
        mlp_body,
        grid=(n // bm,),
        in_specs=[
            pl.BlockSpec((bm, d), lambda i: (i, 0)),
            pl.BlockSpec((d, d), lambda i: (0, 0)),
            pl.BlockSpec((1, d), lambda i: (0, 0)),
            pl.BlockSpec((d, d), lambda i: (0, 0)),
            pl.BlockSpec((1, d), lambda i: (0, 0)),
        ],
        out_specs=pl.BlockSpec((bm, d), lambda i: (i, 0)),
        out_shape=jax.ShapeDtypeStruct((n, d), jnp.float32),
    )(x, W1, b1.reshape(1, d), W2, b2.reshape(1, d))


def kernel(input_ids, emb_table, W1, b1, W2, b2):
    b, s = input_ids.shape
    d = emb_table.shape[1]
    ids = input_ids.reshape(b * s)
    gathered = _sc_gather(ids, emb_table)
    out = _tc_mlp(gathered, W1, b1, W2, b2)
    return out.reshape(b, s, d)
